# Initial kernel scaffold; baseline (speedup 1.0000x reference)
import jax
import jax.numpy as jnp
from jax.experimental import pallas as pl


def kernel(x, edge_src, edge_dst, edge_emb, gamma1, beta1, W_self, b_self, W_m1, b_m1, W_m2, b_m2, W_a1, b_a1, W_a2, b_a2, W_agg, b_agg):
    raise NotImplementedError("write your pallas kernel here")


# speedup vs baseline: 1.0146x; 1.0146x over previous
import jax
import jax.numpy as jnp
from jax.experimental import pallas as pl


def _ln_body(x_ref, g_ref, b_ref, o_ref):
    x = x_ref[...]
    mu = jnp.mean(x, axis=-1, keepdims=True)
    var = jnp.mean((x - mu) ** 2, axis=-1, keepdims=True)
    o_ref[...] = (x - mu) / jnp.sqrt(var + 1e-5) * g_ref[...] + b_ref[...]


def kernel(x, edge_src, edge_dst, edge_emb, gamma1, beta1, W_self, b_self, W_m1, b_m1, W_m2, b_m2, W_a1, b_a1, W_a2, b_a2, W_agg, b_agg):
    n = x.shape[0]
    h = pl.pallas_call(
        _ln_body,
        out_shape=jax.ShapeDtypeStruct(x.shape, x.dtype),
        grid=(50,),
        in_specs=[
            pl.BlockSpec((200, 128), lambda i: (i, 0)),
            pl.BlockSpec((128,), lambda i: (0,)),
            pl.BlockSpec((128,), lambda i: (0,)),
        ],
        out_specs=pl.BlockSpec((200, 128), lambda i: (i, 0)),
    )(x, gamma1, beta1)
    h_src = jnp.take(h, edge_src, axis=0)
    h_dst = jnp.take(h, edge_dst, axis=0)
    edge_context = jnp.concatenate([h_dst, h_src, edge_emb], axis=-1)
    msg_in = jnp.concatenate([h_src, edge_emb], axis=-1)
    gelu = lambda t: jax.nn.gelu(t, approximate=False)
    msg = gelu(msg_in @ W_m1 + b_m1) @ W_m2 + b_m2
    attn_score = (gelu(edge_context @ W_a1 + b_a1) @ W_a2 + b_a2)[:, 0]
    max_scores = jax.ops.segment_max(attn_score, edge_dst, num_segments=n)
    stabilized = attn_score - jnp.take(max_scores, edge_dst, axis=0)
    exp_scores = jnp.exp(stabilized)
    denom = jax.ops.segment_sum(exp_scores, edge_dst, num_segments=n)
    attn_weight = exp_scores / jnp.maximum(jnp.take(denom, edge_dst, axis=0), 1e-12)
    edge_repr = attn_weight[:, None] * msg
    agg = jax.ops.segment_sum(edge_repr, edge_dst, num_segments=n)
    update = (h @ W_self + b_self) + (agg @ W_agg + b_agg)
    out = x + update
    return out, edge_repr


# speedup vs baseline: 3.3096x; 3.2620x over previous
import jax
import jax.numpy as jnp
from jax import lax
from jax.experimental import pallas as pl
from jax.experimental.pallas import tpu as pltpu
from jax.experimental.pallas import tpu_sc as plsc

_N = 10000
_E = 320000
_NC = 2
_NS = 16
_NW = _NC * _NS
_BATCH = 128
_NB = _E // _BATCH
_NP = 10240
_ROWS_PER_TILE = _NP // _NS


def _gelu(t):
    return 0.5 * t * (1.0 + lax.erf(t * 0.7071067811865476))


def _sc_mesh():
    return plsc.VectorSubcoreMesh(
        core_axis_name="c", subcore_axis_name="s",
        num_cores=_NC, num_subcores=_NS)


def _sc_gather(width):
    def body(table_hbm, idx_hbm, out_hbm, idx_v, rows_v, sem):
        w = lax.axis_index("s") * _NC + lax.axis_index("c")
        cnt = (_NB - w + _NW - 1) // _NW

        def step(j, carry):
            base = (w + j * _NW) * _BATCH
            pltpu.sync_copy(idx_hbm.at[pl.ds(base, _BATCH)], idx_v)
            pltpu.async_copy(table_hbm.at[idx_v], rows_v, sem).wait()
            pltpu.sync_copy(rows_v, out_hbm.at[pl.ds(base, _BATCH)])
            return carry

        lax.fori_loop(0, cnt, step, 0)

    return pl.kernel(
        body,
        out_type=jax.ShapeDtypeStruct((_E, width), jnp.float32),
        mesh=_sc_mesh(),
        scratch_types=[
            pltpu.VMEM((_BATCH,), jnp.int32),
            pltpu.VMEM((_BATCH, width), jnp.float32),
            pltpu.SemaphoreType.DMA,
        ],
    )


def _sc_scatter(width):
    def body(vals_hbm, idx_hbm, zeros_hbm, out_hbm, idx_v, rows_v, table_sh):
        c = lax.axis_index("c")
        s = lax.axis_index("s")
        w = s * _NC + c

        pltpu.sync_copy(zeros_hbm, table_sh.at[pl.ds(s * _ROWS_PER_TILE, _ROWS_PER_TILE)])
        plsc.subcore_barrier()

        cnt = (_NB - w + _NW - 1) // _NW

        def step(j, carry):
            base = (w + j * _NW) * _BATCH
            pltpu.sync_copy(idx_hbm.at[pl.ds(base, _BATCH)], idx_v)
            pltpu.sync_copy(vals_hbm.at[pl.ds(base, _BATCH)], rows_v)
            pltpu.sync_copy(rows_v, table_sh.at[idx_v], add=True)
            return carry

        lax.fori_loop(0, cnt, step, 0)
        plsc.subcore_barrier()

        pltpu.sync_copy(table_sh.at[pl.ds(s * _ROWS_PER_TILE, _ROWS_PER_TILE)],
                        out_hbm.at[c, pl.ds(s * _ROWS_PER_TILE, _ROWS_PER_TILE)])

    return pl.kernel(
        body,
        out_type=jax.ShapeDtypeStruct((_NC, _NP, width), jnp.float32),
        mesh=_sc_mesh(),
        scratch_types=[
            pltpu.VMEM((_BATCH,), jnp.int32),
            pltpu.VMEM((_BATCH, width), jnp.float32),
            pltpu.VMEM_SHARED((_NP, width), jnp.float32),
        ],
    )


_SC_CACHE = {}


def _gather_rows(table, idx, width):
    key = ("g", width)
    if key not in _SC_CACHE:
        _SC_CACHE[key] = _sc_gather(width)
    return _SC_CACHE[key](table, idx)


def _scatter_rows(vals, idx, zeros_chunk, width):
    key = ("s", width)
    if key not in _SC_CACHE:
        _SC_CACHE[key] = _sc_scatter(width)
    return _SC_CACHE[key](vals, idx, zeros_chunk)


def _node_body(x_ref, g_ref, b_ref, w_ref, tsrc_ref, tdst_ref, u_ref):
    xb = x_ref[...]
    mu = jnp.mean(xb, axis=-1, keepdims=True)
    var = jnp.mean((xb - mu) ** 2, axis=-1, keepdims=True)
    h = (xb - mu) / jnp.sqrt(var + 1e-5) * g_ref[...] + b_ref[...]
    p = jnp.dot(h, w_ref[...], preferred_element_type=jnp.float32)
    tsrc_ref[...] = p[:, :256]
    tdst_ref[...] = p[:, 256:384]
    u_ref[...] = p[:, 384:512]


def _edge1_body(gs_ref, gd_ref, emb_ref, wme_ref, wae_ref, bm1_ref, ba1_ref,
                wm2_ref, bm2_ref, wa2_ref, ba2_ref, msg_ref, s_ref, gmax_ref):
    gs = gs_ref[...]
    emb = emb_ref[...]
    pre_m = gs[:, :128] + jnp.dot(emb, wme_ref[...], preferred_element_type=jnp.float32) + bm1_ref[...]
    msg_ref[...] = jnp.dot(_gelu(pre_m), wm2_ref[...], preferred_element_type=jnp.float32) + bm2_ref[...]
    pre_a = gs[:, 128:] + gd_ref[...] + jnp.dot(emb, wae_ref[...], preferred_element_type=jnp.float32) + ba1_ref[...]
    s = jnp.sum(_gelu(pre_a) * wa2_ref[...], axis=-1, keepdims=True) + ba2_ref[...]
    s_ref[...] = s
    bm = jnp.max(s, axis=0, keepdims=True)

    @pl.when(pl.program_id(0) == 0)
    def _():
        gmax_ref[...] = bm

    @pl.when(pl.program_id(0) != 0)
    def _():
        gmax_ref[...] = jnp.maximum(gmax_ref[...], bm)


def _exp_body(s_ref, gmax_ref, ew_ref):
    e = jnp.exp(s_ref[...] - gmax_ref[...])
    ew_ref[...] = jnp.broadcast_to(e, (e.shape[0], 128))


def _comb_body(dp_ref, dt_ref):
    dt_ref[...] = dp_ref[0] + dp_ref[1]


def _edge2_body(msg_ref, ew_ref, dg_ref, er_ref):
    wgt = ew_ref[:, :1] / jnp.maximum(dg_ref[:, :1], 1e-12)
    er_ref[...] = msg_ref[...] * wgt


def _final_body(x_ref, u_ref, a0_ref, a1_ref, wagg_ref, bself_ref, bagg_ref, o_ref):
    agg = a0_ref[0] + a1_ref[0]
    o_ref[...] = (x_ref[...] + u_ref[...] + bself_ref[...]
                  + jnp.dot(agg, wagg_ref[...], preferred_element_type=jnp.float32)
                  + bagg_ref[...])


def _tc_call(body, grid, in_specs, out_specs, out_shape):
    return pl.pallas_call(body, grid=grid, in_specs=in_specs,
                          out_specs=out_specs, out_shape=out_shape)


def kernel(x, edge_src, edge_dst, edge_emb, gamma1, beta1, W_self, b_self,
           W_m1, b_m1, W_m2, b_m2, W_a1, b_a1, W_a2, b_a2, W_agg, b_agg):
    f32 = jnp.float32
    Wcat = jnp.concatenate([W_m1[:128], W_a1[128:256], W_a1[:128], W_self], axis=1)
    Wme = W_m1[128:]
    Wae = W_a1[256:]
    bm1 = b_m1.reshape(1, 128)
    ba1 = b_a1.reshape(1, 128)
    bm2 = b_m2.reshape(1, 128)
    wa2 = W_a2.reshape(1, 128)
    ba2 = b_a2.reshape(1, 1)
    bself = b_self.reshape(1, 128)
    bagg = b_agg.reshape(1, 128)
    zeros_chunk = jnp.zeros((_ROWS_PER_TILE, 128), f32)

    BN = 200
    GN = _N // BN
    tsrc, tdst, u = _tc_call(
        _node_body, (GN,),
        [pl.BlockSpec((BN, 128), lambda i: (i, 0)),
         pl.BlockSpec((128,), lambda i: (0,)),
         pl.BlockSpec((128,), lambda i: (0,)),
         pl.BlockSpec((128, 512), lambda i: (0, 0))],
        [pl.BlockSpec((BN, 256), lambda i: (i, 0)),
         pl.BlockSpec((BN, 128), lambda i: (i, 0)),
         pl.BlockSpec((BN, 128), lambda i: (i, 0))],
        [jax.ShapeDtypeStruct((_N, 256), f32),
         jax.ShapeDtypeStruct((_N, 128), f32),
         jax.ShapeDtypeStruct((_N, 128), f32)],
    )(x, gamma1, beta1, Wcat)

    gs = _gather_rows(tsrc, edge_src, 256)
    gd = _gather_rows(tdst, edge_dst, 128)

    BE = 2000
    GE = _E // BE
    msg, s, gmax = _tc_call(
        _edge1_body, (GE,),
        [pl.BlockSpec((BE, 256), lambda i: (i, 0)),
         pl.BlockSpec((BE, 128), lambda i: (i, 0)),
         pl.BlockSpec((BE, 16), lambda i: (i, 0)),
         pl.BlockSpec((16, 128), lambda i: (0, 0)),
         pl.BlockSpec((16, 128), lambda i: (0, 0)),
         pl.BlockSpec((1, 128), lambda i: (0, 0)),
         pl.BlockSpec((1, 128), lambda i: (0, 0)),
         pl.BlockSpec((128, 128), lambda i: (0, 0)),
         pl.BlockSpec((1, 128), lambda i: (0, 0)),
         pl.BlockSpec((1, 128), lambda i: (0, 0)),
         pl.BlockSpec((1, 1), lambda i: (0, 0))],
        [pl.BlockSpec((BE, 128), lambda i: (i, 0)),
         pl.BlockSpec((BE, 1), lambda i: (i, 0)),
         pl.BlockSpec((1, 1), lambda i: (0, 0))],
        [jax.ShapeDtypeStruct((_E, 128), f32),
         jax.ShapeDtypeStruct((_E, 1), f32),
         jax.ShapeDtypeStruct((1, 1), f32)],
    )(gs, gd, edge_emb, Wme, Wae, bm1, ba1, W_m2, bm2, wa2, ba2)

    ew = _tc_call(
        _exp_body, (GE,),
        [pl.BlockSpec((BE, 1), lambda i: (i, 0)),
         pl.BlockSpec((1, 1), lambda i: (0, 0))],
        pl.BlockSpec((BE, 128), lambda i: (i, 0)),
        jax.ShapeDtypeStruct((_E, 128), f32),
    )(s, gmax)

    dpart = _scatter_rows(ew, edge_dst, zeros_chunk, 128)
    BC = 1280
    dtot = _tc_call(
        _comb_body, (_NP // BC,),
        [pl.BlockSpec((_NC, BC, 128), lambda i: (0, i, 0))],
        pl.BlockSpec((BC, 128), lambda i: (i, 0)),
        jax.ShapeDtypeStruct((_NP, 128), f32),
    )(dpart)
    dg = _gather_rows(dtot, edge_dst, 128)

    er = _tc_call(
        _edge2_body, (GE,),
        [pl.BlockSpec((BE, 128), lambda i: (i, 0)),
         pl.BlockSpec((BE, 128), lambda i: (i, 0)),
         pl.BlockSpec((BE, 128), lambda i: (i, 0))],
        pl.BlockSpec((BE, 128), lambda i: (i, 0)),
        jax.ShapeDtypeStruct((_E, 128), f32),
    )(msg, ew, dg)

    apart = _scatter_rows(er, edge_dst, zeros_chunk, 128)

    out = _tc_call(
        _final_body, (GN,),
        [pl.BlockSpec((BN, 128), lambda i: (i, 0)),
         pl.BlockSpec((BN, 128), lambda i: (i, 0)),
         pl.BlockSpec((1, BN, 128), lambda i: (0, i, 0)),
         pl.BlockSpec((1, BN, 128), lambda i: (1, i, 0)),
         pl.BlockSpec((128, 128), lambda i: (0, 0)),
         pl.BlockSpec((1, 128), lambda i: (0, 0)),
         pl.BlockSpec((1, 128), lambda i: (0, 0))],
        pl.BlockSpec((BN, 128), lambda i: (i, 0)),
        jax.ShapeDtypeStruct((_N, 128), f32),
    )(x, u, apart, apart, W_agg, bself, bagg)

    return out, er


# speedup vs baseline: 3.3615x; 1.0157x over previous
import jax
import jax.numpy as jnp
from jax import lax
from jax.experimental import pallas as pl
from jax.experimental.pallas import tpu as pltpu
from jax.experimental.pallas import tpu_sc as plsc

_N = 10000
_E = 320000
_EP = 327680
_NC = 2
_NS = 16
_NW = _NC * _NS
_BATCH = 128
_NB = _E // _BATCH
_NBP = _EP // _BATCH
_NP = 10240
_ROWS_PER_TILE = _NP // _NS


def _gelu(t):
    return 0.5 * t * (1.0 + lax.erf(t * 0.7071067811865476))


def _sc_mesh():
    return plsc.VectorSubcoreMesh(
        core_axis_name="c", subcore_axis_name="s",
        num_cores=_NC, num_subcores=_NS)


def _sc_gather(width):
    def body(table_hbm, idx_hbm, out_hbm, idx_v, rows_v, sem):
        w = lax.axis_index("s") * _NC + lax.axis_index("c")
        cnt = _NBP // _NW

        def step(j, carry):
            base = (w + j * _NW) * _BATCH
            pltpu.sync_copy(idx_hbm.at[pl.ds(base, _BATCH)], idx_v)
            pltpu.async_copy(table_hbm.at[idx_v], rows_v, sem).wait()
            pltpu.sync_copy(rows_v, out_hbm.at[pl.ds(base, _BATCH)])
            return carry

        lax.fori_loop(0, cnt, step, 0)

    return pl.kernel(
        body,
        out_type=jax.ShapeDtypeStruct((_EP, width), jnp.float32),
        mesh=_sc_mesh(),
        scratch_types=[
            pltpu.VMEM((_BATCH,), jnp.int32),
            pltpu.VMEM((_BATCH, width), jnp.float32),
            pltpu.SemaphoreType.DMA,
        ],
    )


def _sc_scatter(width):
    def body(vals_hbm, idx_hbm, zeros_hbm, out_hbm, idx_v, rows_v, table_sh):
        c = lax.axis_index("c")
        s = lax.axis_index("s")
        w = s * _NC + c

        pltpu.sync_copy(zeros_hbm, table_sh.at[pl.ds(s * _ROWS_PER_TILE, _ROWS_PER_TILE)])
        plsc.subcore_barrier()

        cnt = (_NB - w + _NW - 1) // _NW

        def step(j, carry):
            base = (w + j * _NW) * _BATCH
            pltpu.sync_copy(idx_hbm.at[pl.ds(base, _BATCH)], idx_v)
            pltpu.sync_copy(vals_hbm.at[pl.ds(base, _BATCH)], rows_v)
            pltpu.sync_copy(rows_v, table_sh.at[idx_v], add=True)
            return carry

        lax.fori_loop(0, cnt, step, 0)
        plsc.subcore_barrier()

        pltpu.sync_copy(table_sh.at[pl.ds(s * _ROWS_PER_TILE, _ROWS_PER_TILE)],
                        out_hbm.at[c, pl.ds(s * _ROWS_PER_TILE, _ROWS_PER_TILE)])

    return pl.kernel(
        body,
        out_type=jax.ShapeDtypeStruct((_NC, _NP, width), jnp.float32),
        mesh=_sc_mesh(),
        scratch_types=[
            pltpu.VMEM((_BATCH,), jnp.int32),
            pltpu.VMEM((_BATCH, width), jnp.float32),
            pltpu.VMEM_SHARED((_NP, width), jnp.float32),
        ],
    )


def _sc_denom():
    groups = _NBP // _NW // 8

    def body(ew_hbm, idx_hbm, znp_hbm, out_hbm,
             ew8_v, idx8_v, dpriv_v, buf_v, acc_v, stage_sh):
        c = lax.axis_index("c")
        s = lax.axis_index("s")
        w = s * _NC + c

        pltpu.sync_copy(znp_hbm, dpriv_v)

        def grp(g, carry):
            rb = w * (_NBP // _NW) + g * 8
            pltpu.sync_copy(ew_hbm.at[pl.ds(rb, 8)], ew8_v)
            pltpu.sync_copy(idx_hbm.at[pl.ds(rb, 8)], idx8_v)
            for r in range(8):
                for k in range(8):
                    idx16 = idx8_v[r, pl.ds(k * 16, 16)]
                    e16 = ew8_v[r, pl.ds(k * 16, 16)]
                    plsc.addupdate_scatter(dpriv_v, [idx16], e16)
            return carry

        lax.fori_loop(0, groups, grp, 0)

        pltpu.sync_copy(dpriv_v, stage_sh.at[s])
        plsc.subcore_barrier()

        pltpu.sync_copy(stage_sh.at[:, pl.ds(s * _ROWS_PER_TILE, _ROWS_PER_TILE)], buf_v)

        def colsum(k, carry):
            a = buf_v[0, pl.ds(k * 16, 16)]
            for r in range(1, _NS):
                a = a + buf_v[r, pl.ds(k * 16, 16)]
            acc_v[pl.ds(k * 16, 16)] = a
            return carry

        lax.fori_loop(0, _ROWS_PER_TILE // 16, colsum, 0)
        pltpu.sync_copy(acc_v, out_hbm.at[c, pl.ds(s * _ROWS_PER_TILE, _ROWS_PER_TILE)])

    return pl.kernel(
        body,
        out_type=jax.ShapeDtypeStruct((_NC, _NP), jnp.float32),
        mesh=_sc_mesh(),
        scratch_types=[
            pltpu.VMEM((8, 128), jnp.float32),
            pltpu.VMEM((8, 128), jnp.int32),
            pltpu.VMEM((_NP,), jnp.float32),
            pltpu.VMEM((_NS, _ROWS_PER_TILE), jnp.float32),
            pltpu.VMEM((_ROWS_PER_TILE,), jnp.float32),
            pltpu.VMEM_SHARED((_NS, _NP), jnp.float32),
        ],
        compiler_params=pltpu.CompilerParams(needs_layout_passes=False),
    )


def _sc_weight():
    groups = _NBP // _NW // 8

    def body(dp_hbm, ew_hbm, idx_hbm, out_hbm, dtot_v, tmp_v, ew8_v, idx8_v, w8_v):
        c = lax.axis_index("c")
        s = lax.axis_index("s")
        w = s * _NC + c

        pltpu.sync_copy(dp_hbm.at[0], dtot_v)
        pltpu.sync_copy(dp_hbm.at[1], tmp_v)

        def addk(k, carry):
            dtot_v[pl.ds(k * 16, 16)] = dtot_v[pl.ds(k * 16, 16)] + tmp_v[pl.ds(k * 16, 16)]
            return carry

        lax.fori_loop(0, _NP // 16, addk, 0)

        def grp(g, carry):
            rb = w * (_NBP // _NW) + g * 8
            pltpu.sync_copy(ew_hbm.at[pl.ds(rb, 8)], ew8_v)
            pltpu.sync_copy(idx_hbm.at[pl.ds(rb, 8)], idx8_v)
            for r in range(8):
                for k in range(8):
                    idx16 = idx8_v[r, pl.ds(k * 16, 16)]
                    e16 = ew8_v[r, pl.ds(k * 16, 16)]
                    d16 = plsc.load_gather(dtot_v, [idx16])
                    w8_v[r, pl.ds(k * 16, 16)] = e16 / jnp.maximum(d16, 1e-12)
            pltpu.sync_copy(w8_v, out_hbm.at[pl.ds(rb, 8)])
            return carry

        lax.fori_loop(0, groups, grp, 0)

    return pl.kernel(
        body,
        out_type=jax.ShapeDtypeStruct((_NBP, 128), jnp.float32),
        mesh=_sc_mesh(),
        scratch_types=[
            pltpu.VMEM((_NP,), jnp.float32),
            pltpu.VMEM((_NP,), jnp.float32),
            pltpu.VMEM((8, 128), jnp.float32),
            pltpu.VMEM((8, 128), jnp.int32),
            pltpu.VMEM((8, 128), jnp.float32),
        ],
        compiler_params=pltpu.CompilerParams(needs_layout_passes=False),
    )


_SC_CACHE = {}


def _gather_rows(table, idx, width):
    key = ("g", width)
    if key not in _SC_CACHE:
        _SC_CACHE[key] = _sc_gather(width)
    return _SC_CACHE[key](table, idx)


def _scatter_rows(vals, idx, zeros_chunk, width):
    key = ("s", width)
    if key not in _SC_CACHE:
        _SC_CACHE[key] = _sc_scatter(width)
    return _SC_CACHE[key](vals, idx, zeros_chunk)


def _denom_part(ew2d, idx2d, znp):
    if "d" not in _SC_CACHE:
        _SC_CACHE["d"] = _sc_denom()
    return _SC_CACHE["d"](ew2d, idx2d, znp)


def _weight_flat(dpart, ew2d, idx2d):
    if "w" not in _SC_CACHE:
        _SC_CACHE["w"] = _sc_weight()
    return _SC_CACHE["w"](dpart, ew2d, idx2d)


def _node_body(x_ref, g_ref, b_ref, w_ref, tsrc_ref, tdst_ref, u_ref):
    xb = x_ref[...]
    mu = jnp.mean(xb, axis=-1, keepdims=True)
    var = jnp.mean((xb - mu) ** 2, axis=-1, keepdims=True)
    h = (xb - mu) / jnp.sqrt(var + 1e-5) * g_ref[...] + b_ref[...]
    p = jnp.dot(h, w_ref[...], preferred_element_type=jnp.float32)
    tsrc_ref[...] = p[:, :256]
    tdst_ref[...] = p[:, 256:384]
    u_ref[...] = p[:, 384:512]


def _edge1_body(gs_ref, gd_ref, emb_ref, wme_ref, wae_ref, bm1_ref, ba1_ref,
                wm2_ref, bm2_ref, wa2_ref, ba2_ref, msg_ref, s_ref, gmax_ref):
    gs = gs_ref[...]
    emb = emb_ref[...]
    pre_m = gs[:, :128] + jnp.dot(emb, wme_ref[...], preferred_element_type=jnp.float32) + bm1_ref[...]
    msg_ref[...] = jnp.dot(_gelu(pre_m), wm2_ref[...], preferred_element_type=jnp.float32) + bm2_ref[...]
    pre_a = gs[:, 128:] + gd_ref[...] + jnp.dot(emb, wae_ref[...], preferred_element_type=jnp.float32) + ba1_ref[...]
    ga = _gelu(pre_a)
    wa2 = wa2_ref[...]
    rows = [
        lax.dot_general(wa2, ga[c * 128:(c + 1) * 128, :],
                        (((1,), (1,)), ((), ())),
                        preferred_element_type=jnp.float32)
        for c in range(ga.shape[0] // 128)
    ]
    s = jnp.concatenate(rows, axis=0) + ba2_ref[...]
    s_ref[...] = s
    bm = jnp.max(s, axis=(0, 1), keepdims=True)

    @pl.when(pl.program_id(0) == 0)
    def _():
        gmax_ref[...] = bm

    @pl.when(pl.program_id(0) != 0)
    def _():
        gmax_ref[...] = jnp.maximum(gmax_ref[...], bm)


def _exp_body(s_ref, gmax_ref, ew_ref):
    ew_ref[...] = jnp.exp(s_ref[...] - gmax_ref[...])


def _edge2_body(msg_ref, w_ref, ident_ref, er_ref):
    wl = w_ref[...]
    ident = ident_ref[...]
    cols = [
        lax.dot_general(ident, wl[c:c + 1, :], (((1,), (1,)), ((), ())),
                        preferred_element_type=jnp.float32)
        for c in range(wl.shape[0])
    ]
    wcol = jnp.concatenate(cols, axis=0)
    er_ref[...] = msg_ref[...] * wcol


def _final_body(x_ref, u_ref, a0_ref, a1_ref, wagg_ref, bself_ref, bagg_ref, o_ref):
    agg = a0_ref[0] + a1_ref[0]
    o_ref[...] = (x_ref[...] + u_ref[...] + bself_ref[...]
                  + jnp.dot(agg, wagg_ref[...], preferred_element_type=jnp.float32)
                  + bagg_ref[...])


def _tc_call(body, grid, in_specs, out_specs, out_shape):
    return pl.pallas_call(body, grid=grid, in_specs=in_specs,
                          out_specs=out_specs, out_shape=out_shape)


def kernel(x, edge_src, edge_dst, edge_emb, gamma1, beta1, W_self, b_self,
           W_m1, b_m1, W_m2, b_m2, W_a1, b_a1, W_a2, b_a2, W_agg, b_agg):
    f32 = jnp.float32
    Wcat = jnp.concatenate([W_m1[:128], W_a1[128:256], W_a1[:128], W_self], axis=1)
    Wme = W_m1[128:]
    Wae = W_a1[256:]
    bm1 = b_m1.reshape(1, 128)
    ba1 = b_a1.reshape(1, 128)
    bm2 = b_m2.reshape(1, 128)
    wa2 = W_a2.reshape(1, 128)
    ba2 = b_a2.reshape(1, 1)
    bself = b_self.reshape(1, 128)
    bagg = b_agg.reshape(1, 128)
    ident = jnp.eye(128, dtype=f32)
    zeros_chunk = jnp.zeros((_ROWS_PER_TILE, 128), f32)
    znp = jnp.zeros((_NP,), f32)
    pad = _EP - _E
    srcp = jnp.pad(edge_src, (0, pad))
    dstg = jnp.pad(edge_dst, (0, pad))
    dsts = jnp.concatenate([edge_dst, jnp.full((pad,), _NP - 1, jnp.int32)])
    embp = jnp.pad(edge_emb, ((0, pad), (0, 0)))
    idx2d = dsts.reshape(_NBP, 128)

    BN = 200
    GN = _N // BN
    tsrc, tdst, u = _tc_call(
        _node_body, (GN,),
        [pl.BlockSpec((BN, 128), lambda i: (i, 0)),
         pl.BlockSpec((128,), lambda i: (0,)),
         pl.BlockSpec((128,), lambda i: (0,)),
         pl.BlockSpec((128, 512), lambda i: (0, 0))],
        [pl.BlockSpec((BN, 256), lambda i: (i, 0)),
         pl.BlockSpec((BN, 128), lambda i: (i, 0)),
         pl.BlockSpec((BN, 128), lambda i: (i, 0))],
        [jax.ShapeDtypeStruct((_N, 256), f32),
         jax.ShapeDtypeStruct((_N, 128), f32),
         jax.ShapeDtypeStruct((_N, 128), f32)],
    )(x, gamma1, beta1, Wcat)

    gs = _gather_rows(tsrc, srcp, 256)
    gd = _gather_rows(tdst, dstg, 128)

    BE = 4096
    GE = _EP // BE
    SB = BE // 128
    msg, s, gmax = _tc_call(
        _edge1_body, (GE,),
        [pl.BlockSpec((BE, 256), lambda i: (i, 0)),
         pl.BlockSpec((BE, 128), lambda i: (i, 0)),
         pl.BlockSpec((BE, 16), lambda i: (i, 0)),
         pl.BlockSpec((16, 128), lambda i: (0, 0)),
         pl.BlockSpec((16, 128), lambda i: (0, 0)),
         pl.BlockSpec((1, 128), lambda i: (0, 0)),
         pl.BlockSpec((1, 128), lambda i: (0, 0)),
         pl.BlockSpec((128, 128), lambda i: (0, 0)),
         pl.BlockSpec((1, 128), lambda i: (0, 0)),
         pl.BlockSpec((1, 128), lambda i: (0, 0)),
         pl.BlockSpec((1, 1), lambda i: (0, 0))],
        [pl.BlockSpec((BE, 128), lambda i: (i, 0)),
         pl.BlockSpec((SB, 128), lambda i: (i, 0)),
         pl.BlockSpec((1, 1), lambda i: (0, 0))],
        [jax.ShapeDtypeStruct((_EP, 128), f32),
         jax.ShapeDtypeStruct((_NBP, 128), f32),
         jax.ShapeDtypeStruct((1, 1), f32)],
    )(gs, gd, embp, Wme, Wae, bm1, ba1, W_m2, bm2, wa2, ba2)

    BX = 320
    ew2d = _tc_call(
        _exp_body, (_NBP // BX,),
        [pl.BlockSpec((BX, 128), lambda i: (i, 0)),
         pl.BlockSpec((1, 1), lambda i: (0, 0))],
        pl.BlockSpec((BX, 128), lambda i: (i, 0)),
        jax.ShapeDtypeStruct((_NBP, 128), f32),
    )(s, gmax)

    dpart = _denom_part(ew2d, idx2d, znp)
    wflat = _weight_flat(dpart, ew2d, idx2d)

    GE2 = (_E + BE - 1) // BE
    er = _tc_call(
        _edge2_body, (GE2,),
        [pl.BlockSpec((BE, 128), lambda i: (i, 0)),
         pl.BlockSpec((SB, 128), lambda i: (i, 0)),
         pl.BlockSpec((128, 128), lambda i: (0, 0))],
        pl.BlockSpec((BE, 128), lambda i: (i, 0)),
        jax.ShapeDtypeStruct((_E, 128), f32),
    )(msg, wflat, ident)

    apart = _scatter_rows(er, edge_dst, zeros_chunk, 128)

    out = _tc_call(
        _final_body, (GN,),
        [pl.BlockSpec((BN, 128), lambda i: (i, 0)),
         pl.BlockSpec((BN, 128), lambda i: (i, 0)),
         pl.BlockSpec((1, BN, 128), lambda i: (0, i, 0)),
         pl.BlockSpec((1, BN, 128), lambda i: (1, i, 0)),
         pl.BlockSpec((128, 128), lambda i: (0, 0)),
         pl.BlockSpec((1, 128), lambda i: (0, 0)),
         pl.BlockSpec((1, 128), lambda i: (0, 0))],
        pl.BlockSpec((BN, 128), lambda i: (i, 0)),
        jax.ShapeDtypeStruct((_N, 128), f32),
    )(x, u, apart, apart, W_agg, bself, bagg)

    return out, er
